# baseline (speedup 1.0000x reference)
import functools

import jax
import jax.numpy as jnp
from jax.experimental import pallas as pl
from jax.experimental.pallas import tpu as pltpu


def _nn_block_kernel(q_ref, k_ref, md_ref, idx_ref, minv, minb, *, bk, nblk):
    i = pl.program_id(0)

    @pl.when(i == 0)
    def _init():
        minv[...] = jnp.full(minv.shape, jnp.inf, jnp.float32)
        minb[...] = jnp.zeros(minb.shape, jnp.int32)

    q = q_ref[...]
    k = k_ref[...]

    qm2 = q * (-2.0)
    qk = jax.lax.dot_general(
        qm2, k, (((1,), (1,)), ((), ())),
        precision=jax.lax.Precision.DEFAULT,
        preferred_element_type=jnp.float32)
    ksq = jnp.sum(k * k, axis=1)
    dists = qk + ksq[None, :]

    old = minv[...]
    better = dists < old
    minv[...] = jnp.where(better, dists, old)
    minb[...] = jnp.where(better, i, minb[...])

    @pl.when(i == nblk - 1)
    def _finish():
        vals = minv[...]
        gidx = minb[...] * bk + jax.lax.broadcasted_iota(
            jnp.int32, vals.shape, 1)

        w = bk
        while w > 128 and w % 2 == 0:
            h = w // 2
            vl, vr = vals[:, :h], vals[:, h:w]
            il, ir = gidx[:, :h], gidx[:, h:w]
            take_l = (vl < vr) | ((vl == vr) & (il < ir))
            vals = jnp.where(take_l, vl, vr)
            gidx = jnp.where(take_l, il, ir)
            w = h

        bmin = jnp.min(vals, axis=1)
        barg = jnp.min(jnp.where(vals == bmin[:, None], gidx, 0x7FFFFFFF),
                       axis=1)

        qsq = jnp.sum(q * q, axis=1)
        md_ref[...] = qsq + bmin
        idx_ref[...] = barg


@jax.jit
def kernel(queries, keys):
    q, d = queries.shape
    kn, _ = keys.shape
    bk = 8000
    nblk = kn // bk
    assert nblk * bk == kn, (kn, bk)

    out = pl.pallas_call(
        functools.partial(_nn_block_kernel, bk=bk, nblk=nblk),
        grid=(nblk,),
        in_specs=[
            pl.BlockSpec((q, d), lambda i: (0, 0)),
            pl.BlockSpec((bk, d), lambda i: (i, 0)),
        ],
        out_specs=[
            pl.BlockSpec((q,), lambda i: (0,)),
            pl.BlockSpec((q,), lambda i: (0,)),
        ],
        out_shape=[
            jax.ShapeDtypeStruct((q,), jnp.float32),
            jax.ShapeDtypeStruct((q,), jnp.int32),
        ],
        scratch_shapes=[
            pltpu.VMEM((q, bk), jnp.float32),
            pltpu.VMEM((q, bk), jnp.int32),
        ],
        compiler_params=pltpu.CompilerParams(
            dimension_semantics=("arbitrary",),
            needs_layout_passes=True,
        ),
    )(queries, keys)
    min_dist, nn_idx = out
    return min_dist, nn_idx
